# Initial kernel scaffold; baseline (speedup 1.0000x reference)
import jax
import jax.numpy as jnp
from jax.experimental import pallas as pl


def kernel(inp, emb0, emb1, emb2, emb3, proj0, proj1, proj2, proj3):
    raise NotImplementedError("write your pallas kernel here")


# speedup vs baseline: 1.1055x; 1.1055x over previous
import functools

import jax
import jax.numpy as jnp
from jax import lax
from jax.experimental import pallas as pl
from jax.experimental.pallas import tpu as pltpu
from jax.experimental.pallas import tpu_sc as plsc

_CUT = (0, 20000, 40000, 200000, 267735)
_DS = (1024, 256, 64, 16)
_DP = 1024
_NTOK = 8192

_NC = 2
_NS = 16
_NW = _NC * _NS
_TPT = _NTOK // _NW
_CHUNK = 64
_NCHUNK = _TPT // _CHUNK


def _sc_gather(inp_flat, emb0, emb1, emb2, emb3):
    mesh = plsc.VectorSubcoreMesh(core_axis_name="c", subcore_axis_name="s")
    out_type = [jax.ShapeDtypeStruct((_NTOK, d), jnp.float32) for d in _DS]
    scratch_types = (
        [pltpu.VMEM((_TPT,), jnp.int32)]
        + [pltpu.VMEM((_TPT,), jnp.int32) for _ in range(4)]
        + [pltpu.VMEM((_CHUNK, d), jnp.float32) for d in _DS]
        + [pltpu.SemaphoreType.DMA]
    )

    @functools.partial(
        pl.kernel, mesh=mesh, out_type=out_type, scratch_types=scratch_types,
        compiler_params=pltpu.CompilerParams(use_tc_tiling_on_sc=False),
    )
    def k(inp_hbm, e0, e1, e2, e3, x0, x1, x2, x3,
          inp_v, i0, i1, i2, i3, b0, b1, b2, b3, sem):
        embs = (e0, e1, e2, e3)
        xs = (x0, x1, x2, x3)
        idxs = (i0, i1, i2, i3)
        bufs = (b0, b1, b2, b3)
        wid = lax.axis_index("s") * _NC + lax.axis_index("c")
        base = wid * _TPT
        pltpu.sync_copy(inp_hbm.at[pl.ds(base, _TPT)], inp_v)
        for j in range(_TPT // 16):
            v = inp_v[pl.ds(j * 16, 16)]
            for c in range(4):
                lo = _CUT[c]
                sz = _CUT[c + 1] - _CUT[c]
                idxs[c][pl.ds(j * 16, 16)] = jnp.clip(v - lo, 0, sz - 1)
        for ch in range(_NCHUNK):
            cps = [
                pltpu.async_copy(
                    embs[c].at[idxs[c].at[pl.ds(ch * _CHUNK, _CHUNK)]],
                    bufs[c], sem)
                for c in range(4)
            ]
            for cp in cps:
                cp.wait()
            for c in range(4):
                pltpu.sync_copy(
                    bufs[c], xs[c].at[pl.ds(base + ch * _CHUNK, _CHUNK)])

    return k(inp_flat, emb0, emb1, emb2, emb3)


def _tc_matmul(inp2d, x0, x1, x2, x3, p0, p1, p2, p3):
    bm = 256
    grid = (_NTOK // bm,)

    def body(inp_ref, x0r, x1r, x2r, x3r, p0r, p1r, p2r, p3r, o_ref):
        iv = inp_ref[...]
        acc = jnp.zeros((bm, _DP), jnp.float32)
        for c, (xr, pr) in enumerate(
                ((x0r, p0r), (x1r, p1r), (x2r, p2r), (x3r, p3r))):
            m = (iv >= _CUT[c]) & (iv < _CUT[c + 1])
            xc = jnp.where(m, xr[...], 0.0).astype(jnp.bfloat16)
            acc = acc + jnp.dot(xc, pr[...],
                                preferred_element_type=jnp.float32)
        o_ref[...] = acc * (_DP ** 0.5)

    in_specs = (
        [pl.BlockSpec((bm, 1), lambda i: (i, 0))]
        + [pl.BlockSpec((bm, d), lambda i: (i, 0)) for d in _DS]
        + [pl.BlockSpec((d, _DP), lambda i: (0, 0)) for d in _DS]
    )
    return pl.pallas_call(
        body,
        grid=grid,
        in_specs=in_specs,
        out_specs=pl.BlockSpec((bm, _DP), lambda i: (i, 0)),
        out_shape=jax.ShapeDtypeStruct((_NTOK, _DP), jnp.float32),
    )(inp2d, x0, x1, x2, x3, p0, p1, p2, p3)


@jax.jit
def kernel(inp, emb0, emb1, emb2, emb3, proj0, proj1, proj2, proj3):
    inp_flat = inp.reshape(-1)
    xs = _sc_gather(inp_flat, emb0, emb1, emb2, emb3)
    ps = [p.astype(jnp.bfloat16) for p in (proj0, proj1, proj2, proj3)]
    out = _tc_matmul(inp_flat.reshape(-1, 1), *xs, *ps)
    return out.reshape(inp.shape + (_DP,))


# speedup vs baseline: 1.3966x; 1.2633x over previous
import functools

import jax
import jax.numpy as jnp
from jax import lax
from jax.experimental import pallas as pl
from jax.experimental.pallas import tpu as pltpu
from jax.experimental.pallas import tpu_sc as plsc

_CUT = (0, 20000, 40000, 200000, 267735)
_DS = (1024, 256, 64, 16)
_DP = 1024
_NTOK = 8192

_NC = 2
_NS = 16
_NW = _NC * _NS
_TPT = _NTOK // _NW
_CHUNK = 64
_NCHUNK = _TPT // _CHUNK


def _sc_gather(inp_flat, emb0, emb1, emb2, emb3):
    mesh = plsc.VectorSubcoreMesh(core_axis_name="c", subcore_axis_name="s")
    out_type = [jax.ShapeDtypeStruct((_NTOK, d), jnp.float32) for d in _DS]
    scratch_types = (
        [pltpu.VMEM((_TPT,), jnp.int32)]
        + [pltpu.VMEM((_TPT,), jnp.int32) for _ in range(2)]
        + [pltpu.VMEM((_CHUNK, d), jnp.float32) for d in _DS]
        + [pltpu.SemaphoreType.DMA, pltpu.SemaphoreType.DMA]
    )

    @functools.partial(
        pl.kernel, mesh=mesh, out_type=out_type, scratch_types=scratch_types
    )
    def k(inp_hbm, e0, e1, e2, e3, x0, x1, x2, x3,
          inp_v, i0, i1, b0, b1, b2, b3, sem, sem2):
        wid = lax.axis_index("s") * _NC + lax.axis_index("c")
        base = wid * _TPT
        pltpu.sync_copy(inp_hbm.at[pl.ds(base, _TPT)], inp_v)
        for j in range(_TPT // 16):
            v = inp_v[pl.ds(j * 16, 16)]
            for c, iref in ((0, i0), (1, i1)):
                lo = _CUT[c]
                sz = _CUT[c + 1] - _CUT[c]
                iref[pl.ds(j * 16, 16)] = jnp.clip(v - lo, 0, sz - 1)
        for ch in range(_NCHUNK):
            cps = [
                pltpu.async_copy(
                    e0.at[i0.at[pl.ds(ch * _CHUNK, _CHUNK)]], b0, sem),
                pltpu.async_copy(
                    e1.at[i1.at[pl.ds(ch * _CHUNK, _CHUNK)]], b1, sem),
            ]
            rcps = []
            for j in range(_CHUNK // 16):
                v = inp_v[pl.ds(ch * _CHUNK + j * 16, 16)]
                v2 = jnp.clip(v - _CUT[2], 0, _CUT[3] - _CUT[2] - 1)
                v3 = jnp.clip(v - _CUT[3], 0, _CUT[4] - _CUT[3] - 1)
                for l in range(16):
                    r = j * 16 + l
                    rcps.append(pltpu.async_copy(
                        e2.at[pl.ds(v2[l], 1)], b2.at[pl.ds(r, 1)], sem2))
                    rcps.append(pltpu.async_copy(
                        e3.at[pl.ds(v3[l], 1)], b3.at[pl.ds(r, 1)], sem2))
            for cp in cps:
                cp.wait()
            for cp in rcps:
                cp.wait()
            pltpu.sync_copy(b0, x0.at[pl.ds(base + ch * _CHUNK, _CHUNK)])
            pltpu.sync_copy(b1, x1.at[pl.ds(base + ch * _CHUNK, _CHUNK)])
            pltpu.sync_copy(b2, x2.at[pl.ds(base + ch * _CHUNK, _CHUNK)])
            pltpu.sync_copy(b3, x3.at[pl.ds(base + ch * _CHUNK, _CHUNK)])

    return k(inp_flat, emb0, emb1, emb2, emb3)


def _tc_matmul(inp2d, x0, x1, x2, x3, p0, p1, p2, p3):
    bm = 256
    grid = (_NTOK // bm,)

    def body(inp_ref, x0r, x1r, x2r, x3r, p0r, p1r, p2r, p3r, o_ref):
        iv = inp_ref[...]
        acc = jnp.zeros((bm, _DP), jnp.float32)
        for c, (xr, pr) in enumerate(
                ((x0r, p0r), (x1r, p1r), (x2r, p2r), (x3r, p3r))):
            m = (iv >= _CUT[c]) & (iv < _CUT[c + 1])
            xc = jnp.where(m, xr[...], 0.0).astype(jnp.bfloat16)
            acc = acc + jnp.dot(xc, pr[...],
                                preferred_element_type=jnp.float32)
        o_ref[...] = acc * (_DP ** 0.5)

    in_specs = (
        [pl.BlockSpec((bm, 1), lambda i: (i, 0))]
        + [pl.BlockSpec((bm, d), lambda i: (i, 0)) for d in _DS]
        + [pl.BlockSpec((d, _DP), lambda i: (0, 0)) for d in _DS]
    )
    return pl.pallas_call(
        body,
        grid=grid,
        in_specs=in_specs,
        out_specs=pl.BlockSpec((bm, _DP), lambda i: (i, 0)),
        out_shape=jax.ShapeDtypeStruct((_NTOK, _DP), jnp.float32),
    )(inp2d, x0, x1, x2, x3, p0, p1, p2, p3)


@jax.jit
def kernel(inp, emb0, emb1, emb2, emb3, proj0, proj1, proj2, proj3):
    inp_flat = inp.reshape(-1)
    xs = _sc_gather(inp_flat, emb0, emb1, emb2, emb3)
    ps = [p.astype(jnp.bfloat16) for p in (proj0, proj1, proj2, proj3)]
    out = _tc_matmul(inp_flat.reshape(-1, 1), *xs, *ps)
    return out.reshape(inp.shape + (_DP,))


# speedup vs baseline: 2.3152x; 1.6577x over previous
import functools

import jax
import jax.numpy as jnp
from jax import lax
from jax.experimental import pallas as pl
from jax.experimental.pallas import tpu as pltpu
from jax.experimental.pallas import tpu_sc as plsc

_CUT = (0, 20000, 40000, 200000, 267735)
_DS = (1024, 256, 64, 16)
_DP = 1024
_NTOK = 8192

_NC = 2
_NS = 16
_NW = _NC * _NS
_TPT = _NTOK // _NW
_CHUNK = 64
_NCHUNK = _TPT // _CHUNK


def _sc_gather(inp_flat, emb0, emb1, emb2, emb3):
    mesh = plsc.VectorSubcoreMesh(core_axis_name="c", subcore_axis_name="s")
    out_type = [jax.ShapeDtypeStruct((_NTOK, d), jnp.float32) for d in _DS]
    scratch_types = (
        [pltpu.VMEM((_TPT,), jnp.int32)]
        + [pltpu.SemaphoreType.DMA]
    )
    step = 16

    @functools.partial(
        pl.kernel, mesh=mesh, out_type=out_type, scratch_types=scratch_types
    )
    def k(inp_hbm, e0, e1, e2, e3, x0, x1, x2, x3, inp_v, sem):
        embs = (e0, e1, e2, e3)
        xs = (x0, x1, x2, x3)
        wid = lax.axis_index("s") * _NC + lax.axis_index("c")
        base = wid * _TPT
        pltpu.sync_copy(inp_hbm.at[pl.ds(base, _TPT)], inp_v)

        def run(j, fire):
            v = inp_v[pl.ds(j * step, step)]
            for l in range(step):
                t = v[l]
                tok = base + j * step + l
                for c in range(4):
                    @pl.when((t >= _CUT[c]) & (t < _CUT[c + 1]))
                    def _(c=c, t=t, tok=tok):
                        cp = pltpu.make_async_copy(
                            embs[c].at[pl.ds(t - _CUT[c], 1)],
                            xs[c].at[pl.ds(tok, 1)], sem)
                        if fire:
                            cp.start()
                        else:
                            cp.wait()

        pl.loop(0, _TPT // step)(lambda j: run(j, True))
        pl.loop(0, _TPT // step)(lambda j: run(j, False))

    return k(inp_flat, emb0, emb1, emb2, emb3)


def _tc_matmul(inp2d, x0, x1, x2, x3, p0, p1, p2, p3):
    bm = 256
    grid = (_NTOK // bm,)

    def body(inp_ref, x0r, x1r, x2r, x3r, p0r, p1r, p2r, p3r, o_ref):
        iv = inp_ref[...]
        acc = jnp.zeros((bm, _DP), jnp.float32)
        for c, (xr, pr) in enumerate(
                ((x0r, p0r), (x1r, p1r), (x2r, p2r), (x3r, p3r))):
            m = (iv >= _CUT[c]) & (iv < _CUT[c + 1])
            xc = jnp.where(m, xr[...], 0.0).astype(jnp.bfloat16)
            acc = acc + jnp.dot(xc, pr[...],
                                preferred_element_type=jnp.float32)
        o_ref[...] = acc * (_DP ** 0.5)

    in_specs = (
        [pl.BlockSpec((bm, 1), lambda i: (i, 0))]
        + [pl.BlockSpec((bm, d), lambda i: (i, 0)) for d in _DS]
        + [pl.BlockSpec((d, _DP), lambda i: (0, 0)) for d in _DS]
    )
    return pl.pallas_call(
        body,
        grid=grid,
        in_specs=in_specs,
        out_specs=pl.BlockSpec((bm, _DP), lambda i: (i, 0)),
        out_shape=jax.ShapeDtypeStruct((_NTOK, _DP), jnp.float32),
    )(inp2d, x0, x1, x2, x3, p0, p1, p2, p3)


@jax.jit
def kernel(inp, emb0, emb1, emb2, emb3, proj0, proj1, proj2, proj3):
    inp_flat = inp.reshape(-1)
    xs = _sc_gather(inp_flat, emb0, emb1, emb2, emb3)
    ps = [p.astype(jnp.bfloat16) for p in (proj0, proj1, proj2, proj3)]
    out = _tc_matmul(inp_flat.reshape(-1, 1), *xs, *ps)
    return out.reshape(inp.shape + (_DP,))
